# baseline (speedup 1.0000x reference)
import jax
import jax.numpy as jnp
from jax.experimental import pallas as pl
from jax.experimental.pallas import tpu as pltpu

_BR = 400


def _xw_kernel(x_ref, w_ref, o_ref):
    o_ref[...] = jnp.dot(
        x_ref[...], w_ref[...], preferred_element_type=jnp.float32
    ).astype(jnp.bfloat16)


def _layer1_kernel(adj_ref, s1_ref, b1_ref, w2_ref, s2_ref, q_ref, sc_ref):
    a = adj_ref[...]
    m = jnp.maximum(jnp.max(jnp.abs(a)), 1e-30)
    q_ref[0] = jnp.round(a * (127.0 / m)).astype(jnp.int8)
    sc_ref[...] = jnp.full((1, 1, 128), m * (1.0 / 127.0), dtype=jnp.float32)
    h = jnp.dot(a.astype(jnp.bfloat16), s1_ref[...],
                preferred_element_type=jnp.float32)
    h = jnp.maximum(h + b1_ref[...], 0.0)
    s2_ref[...] = jnp.dot(
        h, w2_ref[...].astype(jnp.bfloat16),
        preferred_element_type=jnp.float32).astype(jnp.bfloat16)


def _layer2_kernel(q_ref, sc_ref, s2_ref, b2_ref, o_ref):
    acc = jnp.dot(q_ref[0].astype(jnp.bfloat16), s2_ref[...],
                  preferred_element_type=jnp.float32)
    o_ref[...] = acc * sc_ref[0] + b2_ref[...]


@jax.jit
def kernel(x, edge_index, W1, b1, W2, b2):
    n, d_in = x.shape
    d_hid = W1.shape[1]
    d_out = W2.shape[1]
    adj = edge_index
    nstrip = n // _BR
    grid = (nstrip,)

    s1 = pl.pallas_call(
        _xw_kernel,
        grid=grid,
        in_specs=[
            pl.BlockSpec((_BR, d_in), lambda i: (i, 0)),
            pl.BlockSpec((d_in, d_hid), lambda i: (0, 0)),
        ],
        out_specs=pl.BlockSpec((_BR, d_hid), lambda i: (i, 0)),
        out_shape=jax.ShapeDtypeStruct((n, d_hid), jnp.bfloat16),
    )(x, W1)

    s2, adj_q, scales = pl.pallas_call(
        _layer1_kernel,
        grid=grid,
        in_specs=[
            pl.BlockSpec((_BR, n), lambda i: (i, 0)),
            pl.BlockSpec((n, d_hid), lambda i: (0, 0)),
            pl.BlockSpec((1, d_hid), lambda i: (0, 0)),
            pl.BlockSpec((d_hid, d_out), lambda i: (0, 0)),
        ],
        out_specs=[
            pl.BlockSpec((_BR, d_out), lambda i: (i, 0)),
            pl.BlockSpec((1, _BR, n), lambda i: (i, 0, 0)),
            pl.BlockSpec((1, 1, 128), lambda i: (i, 0, 0)),
        ],
        out_shape=[
            jax.ShapeDtypeStruct((n, d_out), jnp.bfloat16),
            jax.ShapeDtypeStruct((nstrip, _BR, n), jnp.int8),
            jax.ShapeDtypeStruct((nstrip, 1, 128), jnp.float32),
        ],
    )(adj, s1, b1.reshape(1, d_hid), W2)

    out = pl.pallas_call(
        _layer2_kernel,
        grid=grid,
        in_specs=[
            pl.BlockSpec((1, _BR, n), lambda i: (i, 0, 0)),
            pl.BlockSpec((1, 1, 128), lambda i: (i, 0, 0)),
            pl.BlockSpec((n, d_out), lambda i: (0, 0)),
            pl.BlockSpec((1, d_out), lambda i: (0, 0)),
        ],
        out_specs=pl.BlockSpec((_BR, d_out), lambda i: (i, 0)),
        out_shape=jax.ShapeDtypeStruct((n, d_out), jnp.float32),
    )(adj_q, scales, s2, b2.reshape(1, d_out))

    return out


# speedup vs baseline: 1.1742x; 1.1742x over previous
import jax
import jax.numpy as jnp
from jax.experimental import pallas as pl

_BR = 400
_GROUP = 5


def _xw_kernel(x_ref, w_ref, o_ref):
    o_ref[...] = jnp.dot(
        x_ref[...], w_ref[...], preferred_element_type=jnp.float32
    ).astype(jnp.bfloat16)


def _layer1_kernel(adj_ref, s1_ref, b1_ref, w2_ref, s2_ref, q_ref, sc_ref):
    ab = adj_ref[...].astype(jnp.bfloat16)
    mrow = jnp.max(jnp.abs(ab), axis=1, keepdims=True)
    m = jnp.maximum(jnp.max(mrow.astype(jnp.float32)), 1e-30)
    inv = (127.0 / m).astype(jnp.bfloat16)
    q_ref[0] = jnp.round(ab * inv).astype(jnp.int8)
    sc_ref[...] = jnp.full((1, 1, 128), 1.0, dtype=jnp.float32) / inv.astype(
        jnp.float32)
    h = jnp.dot(ab, s1_ref[...], preferred_element_type=jnp.float32)
    h = jnp.maximum(h + b1_ref[...], 0.0)
    s2_ref[...] = jnp.dot(
        h.astype(jnp.bfloat16), w2_ref[...],
        preferred_element_type=jnp.float32).astype(jnp.bfloat16)


def _layer2_kernel(q_ref, sc_ref, s2_ref, b2_ref, o_ref):
    s2 = s2_ref[...]
    b2 = b2_ref[...]
    for p in range(_GROUP):
        acc = jnp.dot(q_ref[p].astype(jnp.bfloat16), s2,
                      preferred_element_type=jnp.float32)
        o_ref[p * _BR:(p + 1) * _BR, :] = acc * sc_ref[p] + b2


@jax.jit
def kernel(x, edge_index, W1, b1, W2, b2):
    n, d_in = x.shape
    d_hid = W1.shape[1]
    d_out = W2.shape[1]
    adj = edge_index
    nstrip = n // _BR

    s1 = pl.pallas_call(
        _xw_kernel,
        grid=(1,),
        in_specs=[
            pl.BlockSpec((n, d_in), lambda i: (0, 0)),
            pl.BlockSpec((d_in, d_hid), lambda i: (0, 0)),
        ],
        out_specs=pl.BlockSpec((n, d_hid), lambda i: (0, 0)),
        out_shape=jax.ShapeDtypeStruct((n, d_hid), jnp.bfloat16),
    )(x, W1.astype(jnp.bfloat16))

    s2, adj_q, scales = pl.pallas_call(
        _layer1_kernel,
        grid=(nstrip,),
        in_specs=[
            pl.BlockSpec((_BR, n), lambda i: (i, 0)),
            pl.BlockSpec((n, d_hid), lambda i: (0, 0)),
            pl.BlockSpec((1, d_hid), lambda i: (0, 0)),
            pl.BlockSpec((d_hid, d_out), lambda i: (0, 0)),
        ],
        out_specs=[
            pl.BlockSpec((_BR, d_out), lambda i: (i, 0)),
            pl.BlockSpec((1, _BR, n), lambda i: (i, 0, 0)),
            pl.BlockSpec((1, 1, 128), lambda i: (i, 0, 0)),
        ],
        out_shape=[
            jax.ShapeDtypeStruct((n, d_out), jnp.bfloat16),
            jax.ShapeDtypeStruct((nstrip, _BR, n), jnp.int8),
            jax.ShapeDtypeStruct((nstrip, 1, 128), jnp.float32),
        ],
    )(adj, s1, b1.reshape(1, d_hid), W2.astype(jnp.bfloat16))

    out = pl.pallas_call(
        _layer2_kernel,
        grid=(nstrip // _GROUP,),
        in_specs=[
            pl.BlockSpec((_GROUP, _BR, n), lambda i: (i, 0, 0)),
            pl.BlockSpec((_GROUP, 1, 128), lambda i: (i, 0, 0)),
            pl.BlockSpec((n, d_out), lambda i: (0, 0)),
            pl.BlockSpec((1, d_out), lambda i: (0, 0)),
        ],
        out_specs=pl.BlockSpec((_GROUP * _BR, d_out), lambda i: (i, 0)),
        out_shape=jax.ShapeDtypeStruct((n, d_out), jnp.float32),
    )(adj_q, scales, s2, b2.reshape(1, d_out))

    return out
